# baseline (speedup 1.0000x reference)
import functools

import jax
import jax.numpy as jnp
from jax import lax
from jax.experimental import pallas as pl
from jax.experimental.pallas import tpu as pltpu
from jax.experimental.pallas import tpu_sc as plsc

NUM_EMBEDDINGS = 1000000
D = 64
BATCH = 4096
HIST = 200
B_FLAT = BATCH * HIST

_info = plsc.get_sparse_core_info()
NC = _info.num_cores
NS = _info.num_subcores
NW = NC * NS

C = 128
N_CHUNKS = B_FLAT // C
CPW = N_CHUNKS // NW


@functools.partial(
    pl.kernel,
    out_type=jax.ShapeDtypeStruct((B_FLAT, D), jnp.float32),
    mesh=plsc.VectorSubcoreMesh(core_axis_name="c", subcore_axis_name="s"),
    scratch_types=[
        pltpu.VMEM((CPW, C), jnp.int32),
        pltpu.VMEM((C, D), jnp.float32),
        pltpu.SemaphoreType.DMA,
    ],
    compiler_params=pltpu.CompilerParams(use_tc_tiling_on_sc=False),
)
def _emb_lookup(idx_hbm, table_hbm, out_hbm, idx_v, rows_v, sem):
    wid = lax.axis_index("s") * NC + lax.axis_index("c")
    chunk_base = wid * CPW
    pltpu.sync_copy(idx_hbm.at[pl.ds(chunk_base, CPW)], idx_v)

    def body(j, carry):
        pltpu.async_copy(table_hbm.at[idx_v.at[j]], rows_v, sem).wait()
        pltpu.sync_copy(rows_v, out_hbm.at[pl.ds((chunk_base + j) * C, C)])
        return carry

    lax.fori_loop(0, CPW, body, 0)


def kernel(token_ids, weight):
    idx = token_ids.reshape(N_CHUNKS, C).astype(jnp.int32)
    out = _emb_lookup(idx, weight)
    return out.reshape(BATCH, HIST, D)


# speedup vs baseline: 1.1163x; 1.1163x over previous
import functools

import jax
import jax.numpy as jnp
from jax import lax
from jax.experimental import pallas as pl
from jax.experimental.pallas import tpu as pltpu
from jax.experimental.pallas import tpu_sc as plsc

NUM_EMBEDDINGS = 1000000
D = 64
BATCH = 4096
HIST = 200
B_FLAT = BATCH * HIST

_info = plsc.get_sparse_core_info()
NC = _info.num_cores
NS = _info.num_subcores
NW = NC * NS

C = 128
N_CHUNKS = B_FLAT // C
CPW = N_CHUNKS // NW
RPW = B_FLAT // NW

GPB = 2
RB = C * GPB
NBUF = 4
NF = RPW // RB
NG = NF // NBUF


@functools.partial(
    pl.kernel,
    out_type=jax.ShapeDtypeStruct((B_FLAT, D), jnp.float32),
    mesh=plsc.VectorSubcoreMesh(core_axis_name="c", subcore_axis_name="s"),
    scratch_types=[
        pltpu.VMEM((CPW, C), jnp.int32),
        pltpu.VMEM((NBUF, RB, D), jnp.float32),
        pltpu.SemaphoreType.DMA((NBUF,)),
        pltpu.SemaphoreType.DMA((NBUF,)),
    ],
    compiler_params=pltpu.CompilerParams(use_tc_tiling_on_sc=False),
)
def _emb_lookup(idx_hbm, table_hbm, out_hbm, idx_v, rows_v, gsem, wsem):
    wid = lax.axis_index("s") * NC + lax.axis_index("c")
    chunk_base = wid * CPW
    row_base = chunk_base * C
    pltpu.sync_copy(idx_hbm.at[pl.ds(chunk_base, CPW)], idx_v)

    def write_copy(f, b):
        return pltpu.make_async_copy(
            rows_v.at[b], out_hbm.at[pl.ds(row_base + f * RB, RB)], wsem.at[b])

    def outer(g, carry):
        f0 = g * NBUF
        gathers = []
        for b in range(NBUF):
            @pl.when(g > 0)
            def _():
                write_copy(f0 - NBUF + b, b).wait()
            fs = []
            for q in range(GPB):
                j = (f0 + b) * GPB + q
                cp = pltpu.make_async_copy(
                    table_hbm.at[idx_v.at[j]],
                    rows_v.at[b].at[pl.ds(q * C, C)],
                    gsem.at[b])
                cp.start()
                fs.append(cp)
            gathers.append(fs)
        for b in range(NBUF):
            for cp in gathers[b]:
                cp.wait()
            write_copy(f0 + b, b).start()
        return carry

    lax.fori_loop(0, NG, outer, 0)
    for b in range(NBUF):
        write_copy(NF - NBUF + b, b).wait()


def kernel(token_ids, weight):
    idx = token_ids.reshape(N_CHUNKS, C).astype(jnp.int32)
    out = _emb_lookup(idx, weight)
    return out.reshape(BATCH, HIST, D)
